# Initial kernel scaffold; baseline (speedup 1.0000x reference)
import functools

import jax
import jax.numpy as jnp
import numpy as np
from jax import lax
from jax.experimental import pallas as pl
from jax.experimental.pallas import tpu as pltpu
from jax.experimental.pallas import tpu_sc as plsc

K = 16
R = 16
D = 128
OUT = 16
NUM_CEL = 2 ** K
B = 16384

NUM_CORES = 2
NUM_SUBCORES = 16

_PACK = np.zeros((R * K, R), dtype=np.float32)
for _r in range(R):
    for _k in range(K):
        _PACK[_r * K + _k, _r] = float(2 ** _k)

_BLK = 2048


def _hash_body(x_ref, w_ref, m_ref, out_ref):
    projT = lax.dot_general(
        w_ref[...], x_ref[...],
        dimension_numbers=(((1,), (1,)), ((), ())),
        preferred_element_type=jnp.float32)
    bits = (projT > 0).astype(jnp.float32)
    hT = lax.dot_general(
        m_ref[...], bits,
        dimension_numbers=(((0,), (0,)), ((), ())),
        preferred_element_type=jnp.float32)
    out_ref[...] = hT.astype(jnp.int32)


def _compute_hashes(X, Wflat, M):
    return pl.pallas_call(
        _hash_body,
        grid=(B // _BLK,),
        in_specs=[
            pl.BlockSpec((_BLK, D), lambda i: (i, 0)),
            pl.BlockSpec((R * K, D), lambda i: (0, 0)),
            pl.BlockSpec((R * K, R), lambda i: (0, 0)),
        ],
        out_specs=pl.BlockSpec((R, _BLK), lambda i: (0, i)),
        out_shape=jax.ShapeDtypeStruct((R, B), jnp.int32),
    )(X, Wflat, M)


_CHUNKS = B // 16
_UNROLL = 4


def _sc_body(sketch, hT, out, row_v, idx_v, acc_v, partner_v, shared):
    c = lax.axis_index("c")
    s = lax.axis_index("s")
    j = s // 2
    o = c * (OUT // 2) + j
    rhalf = s % 2

    def gather_pass(r, first):
        pltpu.sync_copy(sketch.at[o, r], row_v)
        pltpu.sync_copy(hT.at[r], idx_v)

        def body(i, _):
            base = i * (16 * _UNROLL)
            for u in range(_UNROLL):
                sl = pl.ds(base + u * 16, 16)
                idx = idx_v[sl]
                vals = plsc.load_gather(row_v, [idx])
                if first:
                    acc_v[sl] = vals
                else:
                    plsc.addupdate(acc_v.at[sl], vals)
            return 0

        lax.fori_loop(0, _CHUNKS // _UNROLL, body, 0)

    for r8 in range(R // 2):
        gather_pass(rhalf * (R // 2) + r8, first=(r8 == 0))

    @pl.when(rhalf == 1)
    def _publish():
        pltpu.sync_copy(acc_v, shared.at[j])

    plsc.subcore_barrier()

    @pl.when(rhalf == 0)
    def _combine():
        pltpu.sync_copy(shared.at[j], partner_v)

        def body(i, _):
            base = i * (16 * _UNROLL)
            for u in range(_UNROLL):
                sl = pl.ds(base + u * 16, 16)
                acc_v[sl] = (acc_v[sl] + partner_v[sl]) * (1.0 / R)
            return 0

        lax.fori_loop(0, _CHUNKS // _UNROLL, body, 0)
        pltpu.sync_copy(acc_v, out.at[o])


def _sc_gather(sketch, hT):
    mesh = plsc.VectorSubcoreMesh(
        core_axis_name="c", subcore_axis_name="s",
        num_cores=NUM_CORES, num_subcores=NUM_SUBCORES)
    f = pl.kernel(
        _sc_body,
        out_type=jax.ShapeDtypeStruct((OUT, B), jnp.float32),
        mesh=mesh,
        scratch_types=[
            pltpu.VMEM((NUM_CEL,), jnp.float32),
            pltpu.VMEM((B,), jnp.int32),
            pltpu.VMEM((B,), jnp.float32),
            pltpu.VMEM((B,), jnp.float32),
            pltpu.VMEM_SHARED((OUT // 2, B), jnp.float32),
        ],
    )
    return f(sketch, hT)


def kernel(X, W, sketch):
    Wflat = W.reshape(R * K, D)
    M = jnp.asarray(_PACK)
    hT = _compute_hashes(X, Wflat, M)
    predict_t = _sc_gather(sketch, hT)
    return predict_t.T


# speedup vs baseline: 4.2865x; 4.2865x over previous
import functools

import jax
import jax.numpy as jnp
import numpy as np
from jax import lax
from jax.experimental import pallas as pl
from jax.experimental.pallas import tpu as pltpu
from jax.experimental.pallas import tpu_sc as plsc

K = 16
R = 16
D = 128
OUT = 16
NUM_CEL = 2 ** K
B = 16384

NUM_CORES = 2
NUM_SUBCORES = 16

_PACK = np.zeros((R * K, R), dtype=np.float32)
for _r in range(R):
    for _k in range(K):
        _PACK[_r * K + _k, _r] = float(2 ** _k)

_BLK = 2048


def _hash_body(x_ref, w_ref, m_ref, out_ref):
    projT = lax.dot_general(
        w_ref[...], x_ref[...],
        dimension_numbers=(((1,), (1,)), ((), ())),
        preferred_element_type=jnp.float32)
    bits = (projT > 0).astype(jnp.float32)
    hT = lax.dot_general(
        m_ref[...], bits,
        dimension_numbers=(((0,), (0,)), ((), ())),
        preferred_element_type=jnp.float32)
    out_ref[...] = hT.astype(jnp.int32)


def _compute_hashes(X, Wflat, M):
    return pl.pallas_call(
        _hash_body,
        grid=(B // _BLK,),
        in_specs=[
            pl.BlockSpec((_BLK, D), lambda i: (i, 0)),
            pl.BlockSpec((R * K, D), lambda i: (0, 0)),
            pl.BlockSpec((R * K, R), lambda i: (0, 0)),
        ],
        out_specs=pl.BlockSpec((R, _BLK), lambda i: (0, i)),
        out_shape=jax.ShapeDtypeStruct((R, B), jnp.int32),
    )(X, Wflat, M)


_CHUNKS = B // 16
_UNROLL = 4


def _sc_body(sketch, hT, out, row_v, idx_v, acc_v, partner_v, shared):
    c = lax.axis_index("c")
    s = lax.axis_index("s")
    j = s // 2
    o = c * (OUT // 2) + j
    rhalf = s % 2

    def gather_pass(r, first):
        pltpu.sync_copy(sketch.at[o, r], row_v)
        pltpu.sync_copy(hT.at[r], idx_v)

        def body(i, _):
            base = i * (16 * _UNROLL)
            for u in range(_UNROLL):
                sl = pl.ds(base + u * 16, 16)
                idx = idx_v[sl]
                vals = plsc.load_gather(row_v, [idx])
                if first:
                    acc_v[sl] = vals
                else:
                    plsc.addupdate(acc_v.at[sl], vals)
            return 0

        lax.fori_loop(0, _CHUNKS // _UNROLL, body, 0)

    for r8 in range(R // 2):
        gather_pass(rhalf * (R // 2) + r8, first=(r8 == 0))

    @pl.when(rhalf == 1)
    def _publish():
        pltpu.sync_copy(acc_v, shared.at[j])

    plsc.subcore_barrier()

    @pl.when(rhalf == 0)
    def _combine():
        pltpu.sync_copy(shared.at[j], partner_v)

        def body(i, _):
            base = i * (16 * _UNROLL)
            for u in range(_UNROLL):
                sl = pl.ds(base + u * 16, 16)
                acc_v[sl] = (acc_v[sl] + partner_v[sl]) * (1.0 / R)
            return 0

        lax.fori_loop(0, _CHUNKS // _UNROLL, body, 0)
        pltpu.sync_copy(acc_v, out.at[o])


def _sc_gather(sketch, hT):
    mesh = plsc.VectorSubcoreMesh(
        core_axis_name="c", subcore_axis_name="s",
        num_cores=NUM_CORES, num_subcores=NUM_SUBCORES)
    f = pl.kernel(
        _sc_body,
        out_type=jax.ShapeDtypeStruct((OUT, B), jnp.float32),
        mesh=mesh,
        scratch_types=[
            pltpu.VMEM((NUM_CEL,), jnp.float32),
            pltpu.VMEM((B,), jnp.int32),
            pltpu.VMEM((B,), jnp.float32),
            pltpu.VMEM((B,), jnp.float32),
            pltpu.VMEM_SHARED((OUT // 2, B), jnp.float32),
        ],
        compiler_params=pltpu.CompilerParams(needs_layout_passes=False),
    )
    return f(sketch, hT)


def kernel(X, W, sketch):
    Wflat = W.reshape(R * K, D)
    M = jnp.asarray(_PACK)
    hT = _compute_hashes(X, Wflat, M)
    predict_t = _sc_gather(sketch, hT)
    return predict_t.T


# speedup vs baseline: 5.1637x; 1.2047x over previous
import jax
import jax.numpy as jnp
import numpy as np
from jax import lax
from jax.experimental import pallas as pl
from jax.experimental.pallas import tpu as pltpu
from jax.experimental.pallas import tpu_sc as plsc

K = 16
R = 16
D = 128
OUT = 16
NUM_CEL = 2 ** K
B = 16384

NUM_CORES = 2
NUM_SUBCORES = 16

_PACK = np.zeros((R * K, R), dtype=np.float32)
for _r in range(R):
    for _k in range(K):
        _PACK[_r * K + _k, _r] = float(2 ** _k)

_BLK = 2048


def _hash_body(x_ref, w_ref, m_ref, out_ref):
    projT = lax.dot_general(
        w_ref[...], x_ref[...],
        dimension_numbers=(((1,), (1,)), ((), ())),
        preferred_element_type=jnp.float32)
    bits = (projT > 0).astype(jnp.float32)
    hT = lax.dot_general(
        m_ref[...], bits,
        dimension_numbers=(((0,), (0,)), ((), ())),
        preferred_element_type=jnp.float32)
    out_ref[...] = hT.astype(jnp.int32)


def _compute_hashes(X, Wflat, M):
    return pl.pallas_call(
        _hash_body,
        grid=(B // _BLK,),
        in_specs=[
            pl.BlockSpec((_BLK, D), lambda i: (i, 0)),
            pl.BlockSpec((R * K, D), lambda i: (0, 0)),
            pl.BlockSpec((R * K, R), lambda i: (0, 0)),
        ],
        out_specs=pl.BlockSpec((R, _BLK), lambda i: (0, i)),
        out_shape=jax.ShapeDtypeStruct((R, B), jnp.int32),
    )(X, Wflat, M)


_CHUNKS = B // 16
_UNROLL = 8
_NR = R // 2


def _sc_body(sketch, hTf, out, row_v, idx0_v, idx1_v, acc_v, shared,
             row_sem, i0_sem, i1_sem):
    c = lax.axis_index("c")
    s = lax.axis_index("s")
    j = s // 2
    o = c * (OUT // 2) + j
    rhalf = s % 2
    r0 = rhalf * _NR

    idxb = [idx0_v, idx1_v]
    isem = [i0_sem, i1_sem]

    def idx_cp(t):
        return pltpu.make_async_copy(hTf.at[r0 + t], idxb[t % 2], isem[t % 2])

    def row_cp(t):
        return pltpu.make_async_copy(sketch.at[o, r0 + t], row_v, row_sem)

    idx_cp(0).start()
    idx_cp(1).start()
    row_cp(0).start()

    for t in range(_NR):
        idx_cp(t).wait()
        row_cp(t).wait()
        ib = idxb[t % 2]
        first = (t == 0)

        def body(i, _, ib=ib, first=first):
            base = i * (16 * _UNROLL)
            sls = [pl.ds(base + u * 16, 16) for u in range(_UNROLL)]
            idxs = [plsc.bitcast(ib[sl], jnp.int32) for sl in sls]
            vals = [plsc.load_gather(row_v, [ix]) for ix in idxs]
            for sl, v in zip(sls, vals):
                if first:
                    acc_v[sl] = v
                else:
                    plsc.addupdate(acc_v.at[sl], v)
            return 0

        lax.fori_loop(0, _CHUNKS // _UNROLL, body, 0)
        if t + 1 < _NR:
            row_cp(t + 1).start()
        if t + 2 < _NR:
            idx_cp(t + 2).start()

    @pl.when(rhalf == 1)
    def _publish():
        pltpu.sync_copy(acc_v, shared.at[j])

    plsc.subcore_barrier()

    @pl.when(rhalf == 0)
    def _combine():
        pltpu.sync_copy(shared.at[j], idx0_v)

        def body(i, _):
            base = i * (16 * _UNROLL)
            for u in range(_UNROLL):
                sl = pl.ds(base + u * 16, 16)
                acc_v[sl] = (acc_v[sl] + idx0_v[sl]) * (1.0 / R)
            return 0

        lax.fori_loop(0, _CHUNKS // _UNROLL, body, 0)
        pltpu.sync_copy(acc_v, out.at[o])


def _sc_gather(sketch, hT):
    hTf = lax.bitcast_convert_type(hT, jnp.float32)
    mesh = plsc.VectorSubcoreMesh(
        core_axis_name="c", subcore_axis_name="s",
        num_cores=NUM_CORES, num_subcores=NUM_SUBCORES)
    f = pl.kernel(
        _sc_body,
        out_type=jax.ShapeDtypeStruct((OUT, B), jnp.float32),
        mesh=mesh,
        scratch_types=[
            pltpu.VMEM((NUM_CEL,), jnp.float32),
            pltpu.VMEM((B,), jnp.float32),
            pltpu.VMEM((B,), jnp.float32),
            pltpu.VMEM((B,), jnp.float32),
            pltpu.VMEM_SHARED((OUT // 2, B), jnp.float32),
            pltpu.SemaphoreType.DMA,
            pltpu.SemaphoreType.DMA,
            pltpu.SemaphoreType.DMA,
        ],
        compiler_params=pltpu.CompilerParams(needs_layout_passes=False),
    )
    return f(sketch, hTf)


def kernel(X, W, sketch):
    Wflat = W.reshape(R * K, D)
    M = jnp.asarray(_PACK)
    hT = _compute_hashes(X, Wflat, M)
    predict_t = _sc_gather(sketch, hT)
    return predict_t.T


# speedup vs baseline: 5.1697x; 1.0012x over previous
import jax
import jax.numpy as jnp
import numpy as np
from jax import lax
from jax.experimental import pallas as pl
from jax.experimental.pallas import tpu as pltpu
from jax.experimental.pallas import tpu_sc as plsc

K = 16
R = 16
D = 128
OUT = 16
NUM_CEL = 2 ** K
B = 16384

NUM_CORES = 2
NUM_SUBCORES = 16

_PACK = np.zeros((R * K, R), dtype=np.float32)
for _r in range(R):
    for _k in range(K):
        _PACK[_r * K + _k, _r] = float(2 ** _k)

_BLK = 2048


def _hash_body(x_ref, w_ref, m_ref, out_ref):
    projT = lax.dot_general(
        w_ref[...], x_ref[...],
        dimension_numbers=(((1,), (1,)), ((), ())),
        preferred_element_type=jnp.float32)
    bits = (projT > 0).astype(jnp.float32)
    hT = lax.dot_general(
        m_ref[...], bits,
        dimension_numbers=(((0,), (0,)), ((), ())),
        preferred_element_type=jnp.float32)
    out_ref[...] = hT.astype(jnp.int32)


def _compute_hashes(X, Wflat, M):
    return pl.pallas_call(
        _hash_body,
        grid=(B // _BLK,),
        in_specs=[
            pl.BlockSpec((_BLK, D), lambda i: (i, 0)),
            pl.BlockSpec((R * K, D), lambda i: (0, 0)),
            pl.BlockSpec((R * K, R), lambda i: (0, 0)),
        ],
        out_specs=pl.BlockSpec((R, _BLK), lambda i: (0, i)),
        out_shape=jax.ShapeDtypeStruct((R, B), jnp.int32),
    )(X, Wflat, M)


_CHUNKS = B // 16
_UNROLL = 8
_NR = R // 2


def _sc_body(sketch, hTf, out, row_v, idx0_v, idx1_v, acc_v, shared,
             row_sem, i0_sem, i1_sem):
    c = lax.axis_index("c")
    s = lax.axis_index("s")
    j = s // 2
    o = c * (OUT // 2) + j
    rhalf = s % 2
    r0 = rhalf * _NR

    idxb = [idx0_v, idx1_v]
    isem = [i0_sem, i1_sem]

    def idx_cp(t):
        return pltpu.make_async_copy(hTf.at[r0 + t], idxb[t % 2], isem[t % 2])

    def row_cp(t):
        return pltpu.make_async_copy(sketch.at[o, r0 + t], row_v, row_sem)

    idx_cp(0).start()
    idx_cp(1).start()
    row_cp(0).start()

    for t in range(_NR):
        idx_cp(t).wait()
        row_cp(t).wait()
        ib = idxb[t % 2]
        first = (t == 0)

        @plsc.parallel_loop(0, B, step=16, unroll=_UNROLL)
        def _gather_loop(i, ib=ib, first=first):
            sl = pl.ds(i, 16)
            ix = plsc.bitcast(ib[sl], jnp.int32)
            v = plsc.load_gather(row_v, [ix])
            if first:
                acc_v[sl] = v
            else:
                plsc.addupdate(acc_v.at[sl], v)
        if t + 1 < _NR:
            row_cp(t + 1).start()
        if t + 2 < _NR:
            idx_cp(t + 2).start()

    @pl.when(rhalf == 1)
    def _publish():
        pltpu.sync_copy(acc_v, shared.at[j])

    plsc.subcore_barrier()

    @pl.when(rhalf == 0)
    def _combine():
        pltpu.sync_copy(shared.at[j], idx0_v)

        def body(i, _):
            base = i * (16 * _UNROLL)
            for u in range(_UNROLL):
                sl = pl.ds(base + u * 16, 16)
                acc_v[sl] = (acc_v[sl] + idx0_v[sl]) * (1.0 / R)
            return 0

        lax.fori_loop(0, _CHUNKS // _UNROLL, body, 0)
        pltpu.sync_copy(acc_v, out.at[o])


def _sc_gather(sketch, hT):
    hTf = lax.bitcast_convert_type(hT, jnp.float32)
    mesh = plsc.VectorSubcoreMesh(
        core_axis_name="c", subcore_axis_name="s",
        num_cores=NUM_CORES, num_subcores=NUM_SUBCORES)
    f = pl.kernel(
        _sc_body,
        out_type=jax.ShapeDtypeStruct((OUT, B), jnp.float32),
        mesh=mesh,
        scratch_types=[
            pltpu.VMEM((NUM_CEL,), jnp.float32),
            pltpu.VMEM((B,), jnp.float32),
            pltpu.VMEM((B,), jnp.float32),
            pltpu.VMEM((B,), jnp.float32),
            pltpu.VMEM_SHARED((OUT // 2, B), jnp.float32),
            pltpu.SemaphoreType.DMA,
            pltpu.SemaphoreType.DMA,
            pltpu.SemaphoreType.DMA,
        ],
        compiler_params=pltpu.CompilerParams(needs_layout_passes=False),
    )
    return f(sketch, hTf)


def kernel(X, W, sketch):
    Wflat = W.reshape(R * K, D)
    M = jnp.asarray(_PACK)
    hT = _compute_hashes(X, Wflat, M)
    predict_t = _sc_gather(sketch, hT)
    return predict_t.T
